# baseline (speedup 1.0000x reference)
import functools

import jax
import jax.numpy as jnp
from jax import lax
from jax.experimental import pallas as pl
from jax.experimental.pallas import tpu as pltpu
from jax.experimental.pallas import tpu_sc as plsc

BATCH = 16384
HIST = 50
EMBED_DIM = 64
NUM_CORES = 2
NUM_SUBCORES = 16
NUM_WORKERS = NUM_CORES * NUM_SUBCORES
BWIN = BATCH // NUM_WORKERS
CHUNK = 128
SUBT = BWIN // CHUNK
CPW = HIST * SUBT
BTILES = BATCH // CHUNK
SCALE = 8.0
LANES = 16
NBUF = 8
AHEAD = 6
TBUF = 4


VOCAB = 1000000
VCH = 128
VCHUNKS = (VOCAB + VCH - 1) // VCH
TNB = 5
TAH = 4
TPW = 245


@functools.cache
def _build_transpose():
    mesh = plsc.VectorSubcoreMesh(core_axis_name="c", subcore_axis_name="s")

    @functools.partial(
        pl.kernel,
        mesh=mesh,
        out_type=jax.ShapeDtypeStruct((VOCAB, EMBED_DIM), jnp.float32),
        scratch_types=[
            pltpu.VMEM((TNB, EMBED_DIM, VCH), jnp.float32),
            pltpu.VMEM((TNB, VCH, EMBED_DIM), jnp.float32),
            pltpu.SemaphoreType.DMA((TNB,)),
            pltpu.SemaphoreType.DMA((TNB,)),
        ],
        compiler_params=pltpu.CompilerParams(
            use_tc_tiling_on_sc=False, needs_layout_passes=False
        ),
    )
    def _transpose(tabt_hbm, out_hbm, slab_v, obuf_v, lsem, ssem):
        wid = lax.axis_index("s") * NUM_CORES + lax.axis_index("c")

        def v0_of(j):
            cid = j * NUM_WORKERS + wid
            return jnp.minimum(cid * VCH, VOCAB - VCH)

        def load(j, b):
            pltpu.async_copy(
                tabt_hbm.at[:, pl.ds(v0_of(j), VCH)], slab_v.at[b],
                lsem.at[b],
            )

        for b in range(TAH):
            load(b, b)

        lane_ids = [lax.iota(jnp.int32, LANES) + kb * LANES
                    for kb in range(VCH // LANES)]

        def outer(jo):
            for b in range(TNB):
                j = jo + b
                v0 = v0_of(j)

                pltpu.make_async_copy(
                    tabt_hbm.at[:, pl.ds(v0, VCH)], slab_v.at[b], lsem.at[b]
                ).wait()

                @pl.when(j >= TNB)
                def _():
                    pltpu.make_async_copy(
                        obuf_v.at[b],
                        out_hbm.at[pl.ds(v0_of(j - TNB), VCH)],
                        ssem.at[b],
                    ).wait()

                for db in range(EMBED_DIM // LANES):

                    def c_body(c, rot, _b=b, _db=db):
                        dvec = rot + _db * LANES
                        for vb in range(VCH // LANES):
                            v = plsc.load_gather(
                                slab_v.at[_b], [dvec, lane_ids[vb]]
                            )
                            plsc.store_scatter(
                                obuf_v.at[_b], [lane_ids[vb], dvec], v
                            )
                        return (rot + 1) & (LANES - 1)

                    lax.fori_loop(0, LANES, c_body, lane_ids[0])

                pltpu.async_copy(
                    obuf_v.at[b], out_hbm.at[pl.ds(v0, VCH)], ssem.at[b]
                )

                jn = j + TAH

                @pl.when(jn < TPW)
                def _():
                    load(jn, (b + TAH) % TNB)

        pl.loop(0, TPW, step=TNB)(outer)

        for t in range(TNB):
            j = TPW - TNB + t
            pltpu.make_async_copy(
                obuf_v.at[j % TNB],
                out_hbm.at[pl.ds(v0_of(j), VCH)],
                ssem.at[j % TNB],
            ).wait()

    return _transpose


@functools.cache
def _build():
    mesh = plsc.VectorSubcoreMesh(core_axis_name="c", subcore_axis_name="s")

    @functools.partial(
        pl.kernel,
        mesh=mesh,
        out_type=jax.ShapeDtypeStruct(
            (HIST, EMBED_DIM // 8, BTILES, 8, CHUNK), jnp.float32
        ),
        scratch_types=[
            pltpu.VMEM((CPW, CHUNK), jnp.int32),
            pltpu.VMEM((NBUF, CHUNK, EMBED_DIM), jnp.float32),
            pltpu.VMEM((TBUF, 8, 8, CHUNK), jnp.float32),
            pltpu.SemaphoreType.DMA,
            pltpu.SemaphoreType.DMA((NBUF,)),
            pltpu.SemaphoreType.DMA((TBUF,)),
        ],
        compiler_params=pltpu.CompilerParams(
            use_tc_tiling_on_sc=False, needs_layout_passes=False
        ),
    )
    def _gather_scale(xt_hbm, tab_hbm, out_hbm, idx_v, rows_v, tbuf_v,
                      isem, gsem, ssem):
        wid = lax.axis_index("s") * NUM_CORES + lax.axis_index("c")
        b_lo = wid * BWIN

        def stage(g, _):
            h = g // SUBT
            b0 = b_lo + lax.rem(g, SUBT) * CHUNK
            pltpu.async_copy(xt_hbm.at[h, pl.ds(b0, CHUNK)], idx_v.at[g], isem)
            return _

        lax.fori_loop(0, CPW, stage, 0)

        def stage_wait(g, _):
            pltpu.make_async_copy(
                xt_hbm.at[0, pl.ds(0, CHUNK)], idx_v.at[0], isem
            ).wait()
            return _

        lax.fori_loop(0, CPW, stage_wait, 0)

        def gather(g, b):
            pltpu.async_copy(tab_hbm.at[idx_v.at[g]], rows_v.at[b], gsem.at[b])

        for b in range(AHEAD):
            gather(b, b)

        lane_ids = [lax.iota(jnp.int32, LANES) + kb * LANES
                    for kb in range(CHUNK // LANES)]

        def outer(go):
            for b in range(NBUF):
                g = go + b
                h = g // SUBT
                btile = wid * SUBT + lax.rem(g, SUBT)
                tb = b % TBUF

                pltpu.make_async_copy(
                    tab_hbm.at[idx_v.at[g]], rows_v.at[b], gsem.at[b]
                ).wait()

                @pl.when(g >= TBUF)
                def _():
                    gp = g - TBUF
                    pltpu.make_async_copy(
                        tbuf_v.at[tb],
                        out_hbm.at[gp // SUBT, :,
                                   wid * SUBT + lax.rem(gp, SUBT)],
                        ssem.at[tb],
                    ).wait()

                for db in range(EMBED_DIM // LANES):

                    def c_body(c, rot, _b=b, _tb=tb, _db=db):
                        col = rot + _db * LANES
                        dt = col >> 3
                        dl = col & 7
                        for kb in range(CHUNK // LANES):
                            v = plsc.load_gather(
                                rows_v.at[_b], [lane_ids[kb], col]
                            )
                            plsc.store_scatter(
                                tbuf_v.at[_tb],
                                [dt, dl, lane_ids[kb]],
                                v * SCALE,
                            )
                        return (rot + 1) & (LANES - 1)

                    lax.fori_loop(0, LANES, c_body, lane_ids[0])

                pltpu.async_copy(
                    tbuf_v.at[tb], out_hbm.at[h, :, btile], ssem.at[tb]
                )

                gn = g + AHEAD

                @pl.when(gn < CPW)
                def _():
                    gather(gn, (b + AHEAD) % NBUF)

        pl.loop(0, CPW, step=NBUF)(outer)

        for t in range(TBUF):
            g = CPW - TBUF + t
            pltpu.make_async_copy(
                tbuf_v.at[g % TBUF],
                out_hbm.at[g // SUBT, :, wid * SUBT + lax.rem(g, SUBT)],
                ssem.at[g % TBUF],
            ).wait()

    return _gather_scale


def kernel(x, input_embedding):
    xt = x.T
    tab_lin = _build_transpose()(input_embedding.T)
    lin = _build()(xt, tab_lin)
    out = lin.transpose(0, 1, 3, 2, 4).reshape(HIST, EMBED_DIM, BATCH)
    return out.transpose(2, 0, 1)


# speedup vs baseline: 5.2475x; 5.2475x over previous
import functools

import jax
import jax.numpy as jnp
from jax import lax
from jax.experimental import pallas as pl
from jax.experimental.pallas import tpu as pltpu
from jax.experimental.pallas import tpu_sc as plsc

BATCH = 16384
HIST = 50
EMBED_DIM = 64
NUM_CORES = 2
NUM_SUBCORES = 16
NUM_WORKERS = NUM_CORES * NUM_SUBCORES
BWIN = BATCH // NUM_WORKERS
CHUNK = 128
SUBT = BWIN // CHUNK
CPW = HIST * SUBT
BTILES = BATCH // CHUNK
SCALE = 8.0
LANES = 16
NBUF = 8
AHEAD = 7
TBUF = 4


@functools.cache
def _build():
    mesh = plsc.VectorSubcoreMesh(core_axis_name="c", subcore_axis_name="s")

    @functools.partial(
        pl.kernel,
        mesh=mesh,
        out_type=jax.ShapeDtypeStruct(
            (HIST, EMBED_DIM // 8, BTILES, 8, CHUNK), jnp.float32
        ),
        scratch_types=[
            pltpu.VMEM((CPW, CHUNK), jnp.int32),
            pltpu.VMEM((NBUF, CHUNK, EMBED_DIM), jnp.float32),
            pltpu.VMEM((TBUF, 8, 8, CHUNK), jnp.float32),
            pltpu.SemaphoreType.DMA,
            pltpu.SemaphoreType.DMA((NBUF,)),
            pltpu.SemaphoreType.DMA((TBUF,)),
        ],
        compiler_params=pltpu.CompilerParams(
            use_tc_tiling_on_sc=False, needs_layout_passes=False
        ),
    )
    def _gather_scale(xt_hbm, tab_hbm, out_hbm, idx_v, rows_v, tbuf_v,
                      isem, gsem, ssem):
        wid = lax.axis_index("s") * NUM_CORES + lax.axis_index("c")
        b_lo = wid * BWIN

        def stage(g, _):
            h = g // SUBT
            b0 = b_lo + lax.rem(g, SUBT) * CHUNK
            pltpu.async_copy(xt_hbm.at[h, pl.ds(b0, CHUNK)], idx_v.at[g], isem)
            return _

        lax.fori_loop(0, CPW, stage, 0)

        def stage_wait(g, _):
            pltpu.make_async_copy(
                xt_hbm.at[0, pl.ds(0, CHUNK)], idx_v.at[0], isem
            ).wait()
            return _

        lax.fori_loop(0, CPW, stage_wait, 0)

        def gather(g, b):
            pltpu.async_copy(tab_hbm.at[idx_v.at[g]], rows_v.at[b], gsem.at[b])

        for b in range(AHEAD):
            gather(b, b)

        lane_ids = [lax.iota(jnp.int32, LANES) + kb * LANES
                    for kb in range(CHUNK // LANES)]

        def outer(go):
            for b in range(NBUF):
                g = go + b
                h = g // SUBT
                btile = wid * SUBT + lax.rem(g, SUBT)
                tb = b % TBUF

                pltpu.make_async_copy(
                    tab_hbm.at[idx_v.at[g]], rows_v.at[b], gsem.at[b]
                ).wait()

                @pl.when(g >= TBUF)
                def _():
                    gp = g - TBUF
                    pltpu.make_async_copy(
                        tbuf_v.at[tb],
                        out_hbm.at[gp // SUBT, :,
                                   wid * SUBT + lax.rem(gp, SUBT)],
                        ssem.at[tb],
                    ).wait()

                for db in range(EMBED_DIM // LANES):

                    def c_body(c, rot, _b=b, _tb=tb, _db=db):
                        col = rot + _db * LANES
                        dt = col >> 3
                        dl = col & 7
                        for kb in range(CHUNK // LANES):
                            v = plsc.load_gather(
                                rows_v.at[_b], [lane_ids[kb], col]
                            )
                            plsc.store_scatter(
                                tbuf_v.at[_tb],
                                [dt, dl, lane_ids[kb]],
                                v * SCALE,
                            )
                        return (rot + 1) & (LANES - 1)

                    lax.fori_loop(0, LANES, c_body, lane_ids[0])

                pltpu.async_copy(
                    tbuf_v.at[tb], out_hbm.at[h, :, btile], ssem.at[tb]
                )

                gn = g + AHEAD

                @pl.when(gn < CPW)
                def _():
                    gather(gn, (b + AHEAD) % NBUF)

        pl.loop(0, CPW, step=NBUF)(outer)

        for t in range(TBUF):
            g = CPW - TBUF + t
            pltpu.make_async_copy(
                tbuf_v.at[g % TBUF],
                out_hbm.at[g // SUBT, :, wid * SUBT + lax.rem(g, SUBT)],
                ssem.at[g % TBUF],
            ).wait()

    return _gather_scale


def kernel(x, input_embedding):
    xt = x.T
    lin = _build()(xt, input_embedding)
    out = lin.transpose(0, 1, 3, 2, 4).reshape(HIST, EMBED_DIM, BATCH)
    return out.transpose(2, 0, 1)


# speedup vs baseline: 5.2617x; 1.0027x over previous
import functools

import jax
import jax.numpy as jnp
from jax import lax
from jax.experimental import pallas as pl
from jax.experimental.pallas import tpu as pltpu
from jax.experimental.pallas import tpu_sc as plsc

BATCH = 16384
HIST = 50
EMBED_DIM = 64
NUM_CORES = 2
NUM_SUBCORES = 16
NUM_WORKERS = NUM_CORES * NUM_SUBCORES
BWIN = BATCH // NUM_WORKERS
CHUNK = 128
SUBT = BWIN // CHUNK
CPW = HIST * SUBT
BTILES = BATCH // CHUNK
SCALE = 8.0
LANES = 16
NBUF = 4
AHEAD = 3
TBUF = 4


@functools.cache
def _build():
    mesh = plsc.VectorSubcoreMesh(core_axis_name="c", subcore_axis_name="s")

    @functools.partial(
        pl.kernel,
        mesh=mesh,
        out_type=jax.ShapeDtypeStruct(
            (HIST, EMBED_DIM // 8, BTILES, 8, CHUNK), jnp.float32
        ),
        scratch_types=[
            pltpu.VMEM((CPW, CHUNK), jnp.int32),
            pltpu.VMEM((NBUF, CHUNK, EMBED_DIM), jnp.float32),
            pltpu.VMEM((TBUF, 8, 8, CHUNK), jnp.float32),
            pltpu.SemaphoreType.DMA,
            pltpu.SemaphoreType.DMA((NBUF,)),
            pltpu.SemaphoreType.DMA((TBUF,)),
        ],
        compiler_params=pltpu.CompilerParams(
            use_tc_tiling_on_sc=False, needs_layout_passes=False
        ),
    )
    def _gather_scale(xt_hbm, tab_hbm, out_hbm, idx_v, rows_v, tbuf_v,
                      isem, gsem, ssem):
        wid = lax.axis_index("s") * NUM_CORES + lax.axis_index("c")
        b_lo = wid * BWIN

        def stage(g, _):
            h = g // SUBT
            b0 = b_lo + lax.rem(g, SUBT) * CHUNK
            pltpu.async_copy(xt_hbm.at[h, pl.ds(b0, CHUNK)], idx_v.at[g], isem)
            return _

        lax.fori_loop(0, CPW, stage, 0)

        def stage_wait(g, _):
            pltpu.make_async_copy(
                xt_hbm.at[0, pl.ds(0, CHUNK)], idx_v.at[0], isem
            ).wait()
            return _

        lax.fori_loop(0, CPW, stage_wait, 0)

        def gather(g, b):
            pltpu.async_copy(tab_hbm.at[idx_v.at[g]], rows_v.at[b], gsem.at[b])

        for b in range(AHEAD):
            gather(b, b)

        lane_ids = [lax.iota(jnp.int32, LANES) + kb * LANES
                    for kb in range(CHUNK // LANES)]

        def outer(go):
            for b in range(NBUF):
                g = go + b
                h = g // SUBT
                btile = wid * SUBT + lax.rem(g, SUBT)
                tb = b % TBUF

                pltpu.make_async_copy(
                    tab_hbm.at[idx_v.at[g]], rows_v.at[b], gsem.at[b]
                ).wait()

                @pl.when(g >= TBUF)
                def _():
                    gp = g - TBUF
                    pltpu.make_async_copy(
                        tbuf_v.at[tb],
                        out_hbm.at[gp // SUBT, :,
                                   wid * SUBT + lax.rem(gp, SUBT)],
                        ssem.at[tb],
                    ).wait()

                for db in range(EMBED_DIM // LANES):

                    def c_body(c, rot, _b=b, _tb=tb, _db=db):
                        col = rot + _db * LANES
                        dt = col >> 3
                        dl = col & 7
                        for kb in range(CHUNK // LANES):
                            v = plsc.load_gather(
                                rows_v.at[_b], [lane_ids[kb], col]
                            )
                            plsc.store_scatter(
                                tbuf_v.at[_tb],
                                [dt, dl, lane_ids[kb]],
                                v * SCALE,
                            )
                        return (rot + 1) & (LANES - 1)

                    lax.fori_loop(0, LANES, c_body, lane_ids[0])

                pltpu.async_copy(
                    tbuf_v.at[tb], out_hbm.at[h, :, btile], ssem.at[tb]
                )

                gn = g + AHEAD

                @pl.when(gn < CPW)
                def _():
                    gather(gn, (b + AHEAD) % NBUF)

        pl.loop(0, CPW, step=NBUF)(outer)

        for t in range(TBUF):
            g = CPW - TBUF + t
            pltpu.make_async_copy(
                tbuf_v.at[g % TBUF],
                out_hbm.at[g // SUBT, :, wid * SUBT + lax.rem(g, SUBT)],
                ssem.at[g % TBUF],
            ).wait()

    return _gather_scale


def kernel(x, input_embedding):
    xt = x.T
    lin = _build()(xt, input_embedding)
    out = lin.transpose(0, 1, 3, 2, 4).reshape(HIST, EMBED_DIM, BATCH)
    return out.transpose(2, 0, 1)
